# Initial kernel scaffold; baseline (speedup 1.0000x reference)
import jax
import jax.numpy as jnp
from jax.experimental import pallas as pl


def kernel(node_feats, edge_index, W, b, res_W, res_b, gamma, beta):
    raise NotImplementedError("write your pallas kernel here")


# speedup vs baseline: 8.8898x; 8.8898x over previous
import functools

import jax
import jax.numpy as jnp
from jax import lax
from jax.experimental import pallas as pl
from jax.experimental.pallas import tpu as pltpu
from jax.experimental.pallas import tpu_sc as plsc

NC = 2
NS = 16
NW = NC * NS
L = 16


def _sc_mesh():
    return plsc.VectorSubcoreMesh(core_axis_name="c", subcore_axis_name="s")


def _deg_body(nvec, src_hbm, dst_hbm, zeros_hbm, dego_hbm, degi_hbm,
              idx_src, idx_dst, dego_loc, degi_loc):
    c = lax.axis_index("c")
    s = lax.axis_index("s")
    w = s * NC + c
    pltpu.sync_copy(zeros_hbm, dego_loc)
    pltpu.sync_copy(zeros_hbm, degi_loc)
    pltpu.sync_copy(src_hbm.at[w], idx_src)
    pltpu.sync_copy(dst_hbm.at[w], idx_dst)

    def chunk(k, carry):
        row = lax.shift_right_logical(k, 3)
        col = lax.bitwise_and(k, 7) * L
        for idx_ref, loc in ((idx_src, dego_loc), (idx_dst, degi_loc)):
            idx = idx_ref[row, pl.ds(col, L)]
            cnt, last = plsc.scan_count(idx)
            plsc.addupdate_scatter(
                loc,
                [lax.shift_right_logical(idx, 7), lax.bitwise_and(idx, 127)],
                cnt.astype(jnp.float32),
                mask=last,
            )
        return carry

    lax.fori_loop(0, nvec, chunk, 0)
    pltpu.sync_copy(dego_loc, dego_hbm.at[w])
    pltpu.sync_copy(degi_loc, degi_hbm.at[w])


def _msg_body(nchunk, rows_per_tile, h_hbm, src_hbm, dst_hbm, zeros_hbm,
              agg_hbm, idx_src, idx_dst, rows_v, agg_sh, sem):
    c = lax.axis_index("c")
    s = lax.axis_index("s")
    w = s * NC + c
    r0 = s * rows_per_tile
    pltpu.sync_copy(zeros_hbm, agg_sh.at[pl.ds(r0, rows_per_tile)])
    pltpu.sync_copy(src_hbm.at[w], idx_src)
    pltpu.sync_copy(dst_hbm.at[w], idx_dst)
    plsc.subcore_barrier()

    def chunk(j, carry):
        pltpu.async_copy(h_hbm.at[idx_src.at[j]], rows_v, sem).wait()
        pltpu.sync_copy(rows_v, agg_sh.at[idx_dst.at[j]], add=True)
        return carry

    lax.fori_loop(0, nchunk, chunk, 0)
    plsc.subcore_barrier()
    pltpu.sync_copy(agg_sh.at[pl.ds(r0, rows_per_tile)],
                    agg_hbm.at[c, pl.ds(r0, rows_per_tile)])


def _norm_packed(parts):
    deg = jnp.sum(parts, axis=0)
    return jnp.where(deg > 0.0, lax.rsqrt(jnp.maximum(deg, 1.0)), 0.0)


def _norms_body(po_ref, pi_ref, no_ref, ni_ref):
    no_ref[...] = _norm_packed(po_ref[...])
    ni_ref[...] = _norm_packed(pi_ref[...])


def _h_body(no_ref, x_ref, h_ref):
    h_ref[...] = x_ref[...] * no_ref[...]


def _out_body(a0_ref, a1_ref, ni_ref, x_ref, w_ref, b_ref, rw_ref,
              rb_ref, g_ref, be_ref, o_ref):
    n = x_ref.shape[0]
    agg = (a0_ref[...] + a1_ref[...]) * ni_ref[...]
    conv = jnp.dot(agg, w_ref[...], preferred_element_type=jnp.float32)
    conv = jnp.maximum(conv + b_ref[...], 0.0)
    res = jnp.dot(x_ref[...], rw_ref[...], preferred_element_type=jnp.float32)
    res = jnp.maximum(res + rb_ref[...], 0.0)
    out = conv + res
    mean = jnp.sum(out, axis=0, keepdims=True) / n
    cen = out - mean
    var = jnp.sum(cen * cen, axis=0, keepdims=True) / n
    o_ref[...] = cen * lax.rsqrt(var + 1e-5) * g_ref[...] + be_ref[...]


def kernel(node_feats, edge_index, W, b, res_W, res_b, gamma, beta):
    x = node_feats
    n, d = x.shape
    e = edge_index.shape[1]
    assert e % (NW * L) == 0
    e_per_tile = e // NW
    cw = 125 if e_per_tile % 125 == 0 else 128
    assert e_per_tile % cw == 0
    nchunk = e_per_tile // cw

    npad = -(-n // 2048) * 2048
    rows_p = npad // 128
    rows_per_tile = npad // NS

    src = edge_index[0].astype(jnp.int32)
    dst = edge_index[1].astype(jnp.int32)
    nrow = -(-e_per_tile // 128)
    nvec = nrow * 8
    pad = nrow * 128 - e_per_tile

    def _pad128(a):
        a = a.reshape(NW, e_per_tile)
        a = jnp.pad(a, ((0, 0), (0, pad)), constant_values=npad - 1)
        return a.reshape(NW, nrow, 128)

    src16 = _pad128(src)
    dst16 = _pad128(dst)
    src3 = src.reshape(NW, nchunk, cw)
    dst3 = dst.reshape(NW, nchunk, cw)

    f32 = jnp.float32
    zeros_deg = jnp.zeros((rows_p, 128), f32)
    zeros_msg = jnp.zeros((rows_per_tile, d), f32)

    deg_fn = pl.kernel(
        functools.partial(_deg_body, nvec),
        out_type=[jax.ShapeDtypeStruct((NW, rows_p, 128), f32),
                  jax.ShapeDtypeStruct((NW, rows_p, 128), f32)],
        mesh=_sc_mesh(),
        scratch_types=[
            pltpu.VMEM((nrow, 128), jnp.int32),
            pltpu.VMEM((nrow, 128), jnp.int32),
            pltpu.VMEM((rows_p, 128), f32),
            pltpu.VMEM((rows_p, 128), f32),
        ],
        compiler_params=pltpu.CompilerParams(needs_layout_passes=False),
    )
    dego, degi = deg_fn(src16, dst16, zeros_deg)

    norm_o_p, norm_i_p = pl.pallas_call(
        _norms_body,
        out_shape=[jax.ShapeDtypeStruct((rows_p, 128), f32)] * 2,
    )(dego, degi)
    norm_o = norm_o_p.reshape(npad, 1)[:n]
    norm_i = norm_i_p.reshape(npad, 1)[:n]

    h = pl.pallas_call(
        _h_body,
        out_shape=jax.ShapeDtypeStruct((n, d), f32),
    )(norm_o, x)

    msg_fn = pl.kernel(
        functools.partial(_msg_body, nchunk, rows_per_tile),
        out_type=jax.ShapeDtypeStruct((NC, npad, d), f32),
        mesh=_sc_mesh(),
        scratch_types=[
            pltpu.VMEM((nchunk, cw), jnp.int32),
            pltpu.VMEM((nchunk, cw), jnp.int32),
            pltpu.VMEM((cw, d), f32),
            pltpu.VMEM_SHARED((npad, d), f32),
            pltpu.SemaphoreType.DMA,
        ],
    )
    agg = msg_fn(h, src3, dst3, zeros_msg)

    out = pl.pallas_call(
        _out_body,
        out_shape=jax.ShapeDtypeStruct((n, d), f32),
    )(agg[0, :n], agg[1, :n], norm_i, x, W, b, res_W, res_b, gamma, beta)
    return out


# speedup vs baseline: 10.9120x; 1.2275x over previous
import functools

import jax
import jax.numpy as jnp
from jax import lax
from jax.experimental import pallas as pl
from jax.experimental.pallas import tpu as pltpu
from jax.experimental.pallas import tpu_sc as plsc

NC = 2
NS = 16
NW = NC * NS
L = 16


def _sc_mesh():
    return plsc.VectorSubcoreMesh(core_axis_name="c", subcore_axis_name="s")


def _deg_body(nvec, src_hbm, dst_hbm, zeros_hbm, dego_hbm, degi_hbm,
              idx_src, idx_dst, dego_loc, degi_loc):
    c = lax.axis_index("c")
    s = lax.axis_index("s")
    w = s * NC + c
    pltpu.sync_copy(zeros_hbm, dego_loc)
    pltpu.sync_copy(zeros_hbm, degi_loc)
    pltpu.sync_copy(src_hbm.at[w], idx_src)
    pltpu.sync_copy(dst_hbm.at[w], idx_dst)

    def chunk(row, carry):
        for idx_ref, loc in ((idx_src, dego_loc), (idx_dst, degi_loc)):
            for t in range(8):
                idx = idx_ref[row, pl.ds(t * L, L)]
                cnt, last = plsc.scan_count(idx)
                plsc.addupdate_scatter(
                    loc,
                    [lax.shift_right_logical(idx, 7),
                     lax.bitwise_and(idx, 127)],
                    cnt.astype(jnp.float32),
                    mask=last,
                )
        return carry

    lax.fori_loop(0, nvec // 8, chunk, 0)
    pltpu.sync_copy(dego_loc, dego_hbm.at[w])
    pltpu.sync_copy(degi_loc, degi_hbm.at[w])


def _msg_body(nchunk, rows_per_tile, h_hbm, src_hbm, dst_hbm, zeros_hbm,
              agg_hbm, idx_src, idx_dst, rows0, rows1, agg_sh, sem):
    c = lax.axis_index("c")
    s = lax.axis_index("s")
    w = s * NC + c
    r0 = s * rows_per_tile
    pltpu.sync_copy(zeros_hbm, agg_sh.at[pl.ds(r0, rows_per_tile)])
    plsc.subcore_barrier()

    bufs = (rows0, rows1)
    nh = nchunk // 2
    for p in range(2):
        pltpu.sync_copy(src_hbm.at[w, pl.ds(p * nh, nh)], idx_src)
        pltpu.sync_copy(dst_hbm.at[w, pl.ds(p * nh, nh)], idx_dst)
        pltpu.async_copy(h_hbm.at[idx_src.at[0]], rows0, sem)

        def chunk(k, carry):
            for t in range(2):
                j = 2 * k + t
                buf, nxt = bufs[t], bufs[1 - t]
                pltpu.make_async_copy(h_hbm.at[idx_src.at[j]], buf, sem).wait()

                @pl.when(j + 1 < nh)
                def _():
                    pltpu.async_copy(h_hbm.at[idx_src.at[j + 1]], nxt, sem)

                pltpu.sync_copy(buf, agg_sh.at[idx_dst.at[j]], add=True)
            return carry

        lax.fori_loop(0, nh // 2, chunk, 0)
    plsc.subcore_barrier()
    pltpu.sync_copy(agg_sh.at[pl.ds(r0, rows_per_tile)],
                    agg_hbm.at[c, pl.ds(r0, rows_per_tile)])


def _norm_packed(parts):
    deg = jnp.sum(parts, axis=0)
    return jnp.where(deg > 0.0, lax.rsqrt(jnp.maximum(deg, 1.0)), 0.0)


def _norms_body(po_ref, pi_ref, no_ref, ni_ref):
    no_ref[...] = _norm_packed(po_ref[...])
    ni_ref[...] = _norm_packed(pi_ref[...])


def _h_body(no_ref, x_ref, h_ref):
    h_ref[...] = x_ref[...] * no_ref[...]


def _out_body(agg_ref, ni_ref, x_ref, w_ref, b_ref, rw_ref,
              rb_ref, g_ref, be_ref, o_ref):
    n = x_ref.shape[0]
    agg = (agg_ref[0, :n] + agg_ref[1, :n]) * ni_ref[...]
    conv = jnp.dot(agg, w_ref[...], preferred_element_type=jnp.float32)
    conv = jnp.maximum(conv + b_ref[...], 0.0)
    res = jnp.dot(x_ref[...], rw_ref[...], preferred_element_type=jnp.float32)
    res = jnp.maximum(res + rb_ref[...], 0.0)
    out = conv + res
    mean = jnp.sum(out, axis=0, keepdims=True) / n
    cen = out - mean
    var = jnp.sum(cen * cen, axis=0, keepdims=True) / n
    o_ref[...] = cen * lax.rsqrt(var + 1e-5) * g_ref[...] + be_ref[...]


def kernel(node_feats, edge_index, W, b, res_W, res_b, gamma, beta):
    x = node_feats
    n, d = x.shape
    e = edge_index.shape[1]
    assert e % (NW * L) == 0
    e_per_tile = e // NW
    cw = 125 if e_per_tile % 125 == 0 else 128
    assert e_per_tile % cw == 0
    nchunk = e_per_tile // cw

    npad = -(-n // 2048) * 2048
    rows_p = npad // 128
    rows_per_tile = npad // NS

    src = edge_index[0].astype(jnp.int32)
    dst = edge_index[1].astype(jnp.int32)
    nrow = -(-e_per_tile // 128)
    nvec = nrow * 8
    pad = nrow * 128 - e_per_tile

    def _pad128(a):
        a = a.reshape(NW, e_per_tile)
        a = jnp.pad(a, ((0, 0), (0, pad)), constant_values=npad - 1)
        return a.reshape(NW, nrow, 128)

    src16 = _pad128(src)
    dst16 = _pad128(dst)
    src3 = src.reshape(NW, nchunk, cw)
    dst3 = dst.reshape(NW, nchunk, cw)

    f32 = jnp.float32
    zeros_deg = jnp.zeros((rows_p, 128), f32)
    zeros_msg = jnp.zeros((rows_per_tile, d), f32)

    deg_fn = pl.kernel(
        functools.partial(_deg_body, nvec),
        out_type=[jax.ShapeDtypeStruct((NW, rows_p, 128), f32),
                  jax.ShapeDtypeStruct((NW, rows_p, 128), f32)],
        mesh=_sc_mesh(),
        scratch_types=[
            pltpu.VMEM((nrow, 128), jnp.int32),
            pltpu.VMEM((nrow, 128), jnp.int32),
            pltpu.VMEM((rows_p, 128), f32),
            pltpu.VMEM((rows_p, 128), f32),
        ],
        compiler_params=pltpu.CompilerParams(needs_layout_passes=False),
    )
    dego, degi = deg_fn(src16, dst16, zeros_deg)

    norm_o_p, norm_i_p = pl.pallas_call(
        _norms_body,
        out_shape=[jax.ShapeDtypeStruct((rows_p, 128), f32)] * 2,
    )(dego, degi)
    norm_o = norm_o_p.reshape(npad, 1)[:n]
    norm_i = norm_i_p.reshape(npad, 1)[:n]

    h = pl.pallas_call(
        _h_body,
        out_shape=jax.ShapeDtypeStruct((n, d), f32),
    )(norm_o, x)

    msg_fn = pl.kernel(
        functools.partial(_msg_body, nchunk, rows_per_tile),
        out_type=jax.ShapeDtypeStruct((NC, npad, d), f32),
        mesh=_sc_mesh(),
        scratch_types=[
            pltpu.VMEM((nchunk // 2, cw), jnp.int32),
            pltpu.VMEM((nchunk // 2, cw), jnp.int32),
            pltpu.VMEM((cw, d), f32),
            pltpu.VMEM((cw, d), f32),
            pltpu.VMEM_SHARED((npad, d), f32),
            pltpu.SemaphoreType.DMA,
        ],
    )
    agg = msg_fn(h, src3, dst3, zeros_msg)

    out = pl.pallas_call(
        _out_body,
        out_shape=jax.ShapeDtypeStruct((n, d), f32),
    )(agg, norm_i, x, W, b, res_W, res_b, gamma, beta)
    return out


# speedup vs baseline: 10.9326x; 1.0019x over previous
import functools

import jax
import jax.numpy as jnp
from jax import lax
from jax.experimental import pallas as pl
from jax.experimental.pallas import tpu as pltpu
from jax.experimental.pallas import tpu_sc as plsc

NC = 2
NS = 16
NW = NC * NS
L = 16


def _sc_mesh():
    return plsc.VectorSubcoreMesh(core_axis_name="c", subcore_axis_name="s")


def _deg_body(nvec, src_hbm, dst_hbm, zeros_hbm, dego_hbm, degi_hbm,
              idx_src, idx_dst, dego_loc, degi_loc):
    c = lax.axis_index("c")
    s = lax.axis_index("s")
    w = s * NC + c
    pltpu.sync_copy(zeros_hbm, dego_loc)
    pltpu.sync_copy(zeros_hbm, degi_loc)
    pltpu.sync_copy(src_hbm.at[w], idx_src)
    pltpu.sync_copy(dst_hbm.at[w], idx_dst)

    def chunk(row, carry):
        for idx_ref, loc in ((idx_src, dego_loc), (idx_dst, degi_loc)):
            for t in range(8):
                idx = idx_ref[row, pl.ds(t * L, L)]
                cnt, last = plsc.scan_count(idx)
                plsc.addupdate_scatter(
                    loc,
                    [lax.shift_right_logical(idx, 7),
                     lax.bitwise_and(idx, 127)],
                    cnt.astype(jnp.float32),
                    mask=last,
                )
        return carry

    lax.fori_loop(0, nvec // 8, chunk, 0)
    pltpu.sync_copy(dego_loc, dego_hbm.at[w])
    pltpu.sync_copy(degi_loc, degi_hbm.at[w])


def _msg_body(nchunk, rows_per_tile, h_hbm, src_hbm, dst_hbm, zeros_hbm,
              agg_hbm, idx_src, idx_dst, rows0, rows1, agg_sh, gsem, ssem):
    c = lax.axis_index("c")
    s = lax.axis_index("s")
    w = s * NC + c
    r0 = s * rows_per_tile
    pltpu.sync_copy(zeros_hbm, agg_sh.at[pl.ds(r0, rows_per_tile)])
    plsc.subcore_barrier()

    bufs = (rows0, rows1)
    nh = nchunk // 2
    for p in range(2):
        pltpu.sync_copy(src_hbm.at[w, pl.ds(p * nh, nh)], idx_src)
        pltpu.sync_copy(dst_hbm.at[w, pl.ds(p * nh, nh)], idx_dst)
        pltpu.async_copy(h_hbm.at[idx_src.at[0]], rows0, gsem)

        def chunk(k, carry):
            for t in range(2):
                j = 2 * k + t
                buf, nxt = bufs[t], bufs[1 - t]
                pltpu.make_async_copy(h_hbm.at[idx_src.at[j]], buf,
                                      gsem).wait()

                @pl.when(j + 1 < nh)
                def _prefetch():
                    pltpu.async_copy(h_hbm.at[idx_src.at[j + 1]], nxt, gsem)

                pltpu.async_copy(buf, agg_sh.at[idx_dst.at[j]], ssem,
                                 add=True).wait()
            return carry

        lax.fori_loop(0, nh // 2, chunk, 0)
    plsc.subcore_barrier()
    pltpu.sync_copy(agg_sh.at[pl.ds(r0, rows_per_tile)],
                    agg_hbm.at[c, pl.ds(r0, rows_per_tile)])


def _norm_packed(parts):
    deg = jnp.sum(parts, axis=0)
    return jnp.where(deg > 0.0, lax.rsqrt(jnp.maximum(deg, 1.0)), 0.0)


def _norms_body(po_ref, pi_ref, no_ref, ni_ref):
    no_ref[...] = _norm_packed(po_ref[...])
    ni_ref[...] = _norm_packed(pi_ref[...])


def _h_body(no_ref, x_ref, h_ref):
    h_ref[...] = x_ref[...] * no_ref[...]


def _out_body(agg_ref, ni_ref, x_ref, w_ref, b_ref, rw_ref,
              rb_ref, g_ref, be_ref, o_ref):
    n = x_ref.shape[0]
    agg = (agg_ref[0, :n] + agg_ref[1, :n]) * ni_ref[...]
    conv = jnp.dot(agg, w_ref[...], preferred_element_type=jnp.float32)
    conv = jnp.maximum(conv + b_ref[...], 0.0)
    res = jnp.dot(x_ref[...], rw_ref[...], preferred_element_type=jnp.float32)
    res = jnp.maximum(res + rb_ref[...], 0.0)
    out = conv + res
    mean = jnp.sum(out, axis=0, keepdims=True) / n
    cen = out - mean
    var = jnp.sum(cen * cen, axis=0, keepdims=True) / n
    o_ref[...] = cen * lax.rsqrt(var + 1e-5) * g_ref[...] + be_ref[...]


def kernel(node_feats, edge_index, W, b, res_W, res_b, gamma, beta):
    x = node_feats
    n, d = x.shape
    e = edge_index.shape[1]
    assert e % (NW * L) == 0
    e_per_tile = e // NW
    cw = 125 if e_per_tile % 125 == 0 else 128
    assert e_per_tile % cw == 0
    nchunk = e_per_tile // cw

    npad = -(-n // 2048) * 2048
    rows_p = npad // 128
    rows_per_tile = npad // NS

    src = edge_index[0].astype(jnp.int32)
    dst = edge_index[1].astype(jnp.int32)
    nrow = -(-e_per_tile // 128)
    nvec = nrow * 8
    pad = nrow * 128 - e_per_tile

    def _pad128(a):
        a = a.reshape(NW, e_per_tile)
        a = jnp.pad(a, ((0, 0), (0, pad)), constant_values=npad - 1)
        return a.reshape(NW, nrow, 128)

    src16 = _pad128(src)
    dst16 = _pad128(dst)
    src3 = src.reshape(NW, nchunk, cw)
    dst3 = dst.reshape(NW, nchunk, cw)

    f32 = jnp.float32
    zeros_deg = jnp.zeros((rows_p, 128), f32)
    zeros_msg = jnp.zeros((rows_per_tile, d), f32)

    deg_fn = pl.kernel(
        functools.partial(_deg_body, nvec),
        out_type=[jax.ShapeDtypeStruct((NW, rows_p, 128), f32),
                  jax.ShapeDtypeStruct((NW, rows_p, 128), f32)],
        mesh=_sc_mesh(),
        scratch_types=[
            pltpu.VMEM((nrow, 128), jnp.int32),
            pltpu.VMEM((nrow, 128), jnp.int32),
            pltpu.VMEM((rows_p, 128), f32),
            pltpu.VMEM((rows_p, 128), f32),
        ],
        compiler_params=pltpu.CompilerParams(needs_layout_passes=False),
    )
    dego, degi = deg_fn(src16, dst16, zeros_deg)

    norm_o_p, norm_i_p = pl.pallas_call(
        _norms_body,
        out_shape=[jax.ShapeDtypeStruct((rows_p, 128), f32)] * 2,
    )(dego, degi)
    norm_o = norm_o_p.reshape(npad, 1)[:n]
    norm_i = norm_i_p.reshape(npad, 1)[:n]

    h = pl.pallas_call(
        _h_body,
        out_shape=jax.ShapeDtypeStruct((n, d), f32),
    )(norm_o, x)

    msg_fn = pl.kernel(
        functools.partial(_msg_body, nchunk, rows_per_tile),
        out_type=jax.ShapeDtypeStruct((NC, npad, d), f32),
        mesh=_sc_mesh(),
        scratch_types=[
            pltpu.VMEM((nchunk // 2, cw), jnp.int32),
            pltpu.VMEM((nchunk // 2, cw), jnp.int32),
            pltpu.VMEM((cw, d), f32),
            pltpu.VMEM((cw, d), f32),
            pltpu.VMEM_SHARED((npad, d), f32),
            pltpu.SemaphoreType.DMA,
            pltpu.SemaphoreType.DMA,
        ],
    )
    agg = msg_fn(h, src3, dst3, zeros_msg)

    out = pl.pallas_call(
        _out_body,
        out_shape=jax.ShapeDtypeStruct((n, d), f32),
    )(agg, norm_i, x, W, b, res_W, res_b, gamma, beta)
    return out
